# baseline (speedup 1.0000x reference)
import jax
import jax.numpy as jnp
import numpy as np
from jax import lax
from jax.experimental import pallas as pl
from jax.experimental.pallas import tpu as pltpu

_TB = 256


def _conv1_tables():
    hl = np.arange(4).reshape(4, 1, 1, 1, 1, 1)
    co = np.arange(5).reshape(1, 5, 1, 1, 1, 1)
    wp = np.arange(2).reshape(1, 1, 2, 1, 1, 1)
    ws = np.arange(16).reshape(1, 1, 1, 16, 1, 1)
    rw = np.arange(8).reshape(1, 1, 1, 1, 8, 1)
    c = np.arange(28).reshape(1, 1, 1, 1, 1, 28)
    idxs, masks = [], []
    for b in range(7):
        s = min(max(4 * b - 2, 0), 20)
        kh = (s + rw) - (4 * b + hl) + 2
        kw = c - (2 * (ws - 1) + wp) + 2
        valid = (kh >= 0) & (kh < 5) & (kw >= 0) & (kw < 5) & (ws >= 1) & (ws <= 14)
        idx = co * 25 + np.clip(kh, 0, 4) * 5 + np.clip(kw, 0, 4)
        shape = (4, 5, 2, 16, 8, 28)
        idxs.append(np.broadcast_to(idx, shape).reshape(640, 224))
        masks.append(np.broadcast_to(valid, shape).reshape(640, 224))
    return (np.stack(idxs).astype(np.int32),
            np.stack(masks).astype(np.float32))


def _conv2_tables():
    hp = np.arange(8).reshape(8, 1, 1, 1, 1, 1, 1, 1)
    hpar = np.arange(2).reshape(1, 2, 1, 1, 1, 1, 1, 1)
    co = np.arange(3).reshape(1, 1, 3, 1, 1, 1, 1, 1)
    wp = np.arange(2).reshape(1, 1, 1, 2, 1, 1, 1, 1)
    ws = np.arange(8).reshape(1, 1, 1, 1, 8, 1, 1, 1)
    hs = np.arange(16).reshape(1, 1, 1, 1, 1, 16, 1, 1)
    ci = np.arange(5).reshape(1, 1, 1, 1, 1, 1, 5, 1)
    wc = np.arange(16).reshape(1, 1, 1, 1, 1, 1, 1, 16)
    H = 2 * hp + hpar
    w = 2 * ws + wp
    kh = hs - H
    kw = wc - w
    valid = ((kh >= 0) & (kh < 3) & (kw >= 0) & (kw < 3)
             & (H <= 13) & (w <= 13))
    idx = co * 45 + ci * 9 + np.clip(kh, 0, 2) * 3 + np.clip(kw, 0, 2)
    shape = (8, 2, 3, 2, 8, 16, 5, 16)
    return (np.broadcast_to(idx, shape).reshape(768, 1280).astype(np.int32),
            np.broadcast_to(valid, shape).reshape(768, 1280).astype(np.float32))


_A1_IDX, _A1_MASK = _conv1_tables()
_A2_IDX, _A2_MASK = _conv2_tables()
_BP1_MCO = np.broadcast_to(np.arange(5).reshape(1, 5, 1), (2, 5, 16)).reshape(160)
_BP1_MASK = np.broadcast_to(
    ((np.arange(16) >= 1) & (np.arange(16) <= 14)).reshape(1, 1, 16),
    (2, 5, 16)).reshape(160).astype(np.float32)
_BP2_MCO = np.broadcast_to(np.arange(3).reshape(1, 3, 1), (8, 3, 8)).reshape(192)
_BP2_MASK = np.broadcast_to(
    (np.arange(8).reshape(8, 1, 1) <= 6) & (np.arange(8).reshape(1, 1, 8) <= 6),
    (8, 3, 8)).reshape(192).astype(np.float32)


def _fused_body(x_ref, a1_ref, a2_ref, bp1_ref, bp2_ref, wa_ref, bfc1_ref,
                wo_ref, bout_ref, out_ref, p1s_ref):
    f32 = jnp.float32
    bf16 = jnp.bfloat16
    xb = x_ref[...].astype(bf16)

    p1s_ref[0] = jnp.zeros((5, 16, _TB), bf16)
    p1s_ref[15] = jnp.zeros((5, 16, _TB), bf16)

    for b in range(7):
        s = min(max(4 * b - 2, 0), 20)
        xs = xb[:, 28 * s: 28 * s + 224]
        r = lax.dot_general(a1_ref[b], xs, (((1,), (1,)), ((), ())),
                            preferred_element_type=f32)
        v = r.reshape(2, 2, 160, _TB)
        hpool = jnp.maximum(v[:, 0], v[:, 1])
        wv = hpool.reshape(2, 5, 2, 16, _TB)
        p = jnp.maximum(wv[:, :, 0], wv[:, :, 1]).reshape(160, _TB)
        p = jnp.maximum(p + bp1_ref[...], 0.0)
        p1s_ref[pl.ds(1 + 2 * b, 2)] = p.reshape(2, 5, 16, _TB).astype(bf16)

    p1v = p1s_ref[...].reshape(1280, _TB)
    r2 = lax.dot_general(a2_ref[...], p1v, (((1,), (0,)), ((), ())),
                         preferred_element_type=f32)
    v2 = r2.reshape(8, 2, 48, _TB)
    h2 = jnp.maximum(v2[:, 0], v2[:, 1])
    wv2 = h2.reshape(8, 3, 2, 8, _TB)
    p2 = jnp.maximum(wv2[:, :, 0], wv2[:, :, 1]).reshape(192, _TB)
    fv = jnp.maximum(p2 + bp2_ref[...], 0.0).astype(bf16)

    h1 = jnp.dot(wa_ref[...], fv, preferred_element_type=f32) + bfc1_ref[...]
    h1 = jnp.maximum(h1, 0.0).astype(bf16)
    z = jnp.dot(wo_ref[...], h1, preferred_element_type=f32) + bout_ref[...]
    z = z - jnp.max(z, axis=0, keepdims=True)
    e = jnp.exp(z)
    out_ref[...] = e / jnp.sum(e, axis=0, keepdims=True)


def kernel(w1f, b1, w2f, b2, wfc1p, bfc1, wout, bout, x):
    n = x.shape[0]
    n_pad = pl.cdiv(n, _TB) * _TB
    x2 = x.reshape(n, 784).astype(jnp.float32)
    if n_pad != n:
        x2 = jnp.pad(x2, ((0, n_pad - n), (0, 0)))

    bf16 = jnp.bfloat16
    a1 = (jnp.take(w1f, jnp.asarray(_A1_IDX)) * jnp.asarray(_A1_MASK)).astype(bf16)
    a2 = (jnp.take(w2f, jnp.asarray(_A2_IDX)) * jnp.asarray(_A2_MASK)).astype(bf16)
    bp1 = jnp.broadcast_to(
        (jnp.take(b1, jnp.asarray(_BP1_MCO)) * jnp.asarray(_BP1_MASK))[:, None],
        (160, _TB))
    bp2 = jnp.broadcast_to(
        (jnp.take(b2, jnp.asarray(_BP2_MCO)) * jnp.asarray(_BP2_MASK))[:, None],
        (192, _TB))
    w4 = wfc1p.reshape(50, 3, 7, 8)[:, :, :, :7]
    wa = jnp.pad(w4.transpose(0, 2, 1, 3),
                 ((0, 0), (0, 1), (0, 0), (0, 1))).reshape(50, 192).astype(bf16)
    bfc1b = jnp.broadcast_to(bfc1, (50, _TB))
    boutb = jnp.broadcast_to(bout, (10, _TB))
    wo = wout.astype(bf16)

    out = pl.pallas_call(
        _fused_body,
        out_shape=jax.ShapeDtypeStruct((10, n_pad), jnp.float32),
        grid=(n_pad // _TB,),
        in_specs=[
            pl.BlockSpec((_TB, 784), lambda i: (i, 0)),
            pl.BlockSpec((7, 640, 224), lambda i: (0, 0, 0)),
            pl.BlockSpec((768, 1280), lambda i: (0, 0)),
            pl.BlockSpec((160, _TB), lambda i: (0, 0)),
            pl.BlockSpec((192, _TB), lambda i: (0, 0)),
            pl.BlockSpec((50, 192), lambda i: (0, 0)),
            pl.BlockSpec((50, _TB), lambda i: (0, 0)),
            pl.BlockSpec((10, 50), lambda i: (0, 0)),
            pl.BlockSpec((10, _TB), lambda i: (0, 0)),
        ],
        out_specs=pl.BlockSpec((10, _TB), lambda i: (0, i)),
        scratch_shapes=[pltpu.VMEM((16, 5, 16, _TB), bf16)],
        compiler_params=pltpu.CompilerParams(
            dimension_semantics=("parallel",),
            vmem_limit_bytes=64 * 1024 * 1024,
        ),
    )(x2, a1, a2, bp1, bp2, wa, bfc1b, wo, boutb)

    return out[:, :n].T
